# Initial kernel scaffold; baseline (speedup 1.0000x reference)
import jax
import jax.numpy as jnp
from jax.experimental import pallas as pl


def kernel(x, W):
    raise NotImplementedError("write your pallas kernel here")


# speedup vs baseline: 11.2398x; 11.2398x over previous
import jax
import jax.numpy as jnp
from jax.experimental import pallas as pl
from functools import partial

_NIN = 512
_NOUT = 8
_NDISC = 3
_OUT_STD = 0.1
_LO = -0.1
_HI = 0.1


def _mu_kernel(x_ref, wt_ref, o_ref):
    w = wt_ref[...]
    w0 = w[0]
    a = w[1] - w0
    b = w[2] - w[1]
    base = jnp.sum(w0, axis=0, keepdims=True)
    x = x_ref[...]
    m1 = (x > _LO).astype(jnp.float32)
    m2 = (x > _HI).astype(jnp.float32)
    dot = partial(jax.lax.dot_general,
                  dimension_numbers=(((1,), (0,)), ((), ())),
                  preferred_element_type=jnp.float32)
    o_ref[...] = dot(m1, a) + dot(m2, b) + base


def kernel(x, W):
    batch = x.shape[0]
    tile = 2048
    wt = jnp.transpose(W.reshape(_NOUT, _NIN, _NDISC), (2, 1, 0))
    mu = pl.pallas_call(
        _mu_kernel,
        grid=(batch // tile,),
        in_specs=[
            pl.BlockSpec((tile, _NIN), lambda i: (i, 0)),
            pl.BlockSpec((_NDISC, _NIN, _NOUT), lambda i: (0, 0, 0)),
        ],
        out_specs=pl.BlockSpec((tile, _NOUT), lambda i: (i, 0)),
        out_shape=jax.ShapeDtypeStruct((batch, _NOUT), jnp.float32),
    )(x, wt)
    idx = jnp.arange(_NOUT)
    scale_tril = (jnp.zeros((1, _NOUT, _NOUT), dtype=jnp.float32)
                  .at[:, idx, idx].set(_OUT_STD))
    return mu, scale_tril
